# Initial kernel scaffold; baseline (speedup 1.0000x reference)
import jax
import jax.numpy as jnp
from jax.experimental import pallas as pl


def kernel(x, edge_index, W1, b1, W2, b2, W3, b3):
    raise NotImplementedError("write your pallas kernel here")


# speedup vs baseline: 5.4538x; 5.4538x over previous
import jax
import jax.numpy as jnp
from jax import lax
from jax.experimental import pallas as pl
from jax.experimental.pallas import tpu as pltpu
from jax.experimental.pallas import tpu_sc as plsc

N = 10000
E = 320000
D = 128
NW = 32
EPW = E // NW
C = 80
NCHUNK = EPW // C
NP = 10240
RPT = NP // 16
ZB = 128

_SELU_ALPHA = 1.6732632423543772
_SELU_SCALE = 1.0507009873554805


def _sc_aggregate_body(x, src, dst, agg_out, deg_out,
                       idx_s, idx_d, rows, zbuf, hist, agg_sh, sem):
    c = lax.axis_index("c")
    s = lax.axis_index("s")
    wid = c * 16 + s
    ones16 = jnp.ones((16,), jnp.float32)
    zeros16 = jnp.zeros((16,), jnp.float32)

    def zero_hist(i, carry):
        hist[pl.ds(pl.multiple_of(i * 16, 16), 16)] = zeros16
        return carry

    lax.fori_loop(0, NP // 16, zero_hist, 0)

    def zero_row(i, carry):
        for j in range(D // 16):
            zbuf[i, pl.ds(j * 16, 16)] = zeros16
        return carry

    lax.fori_loop(0, ZB, zero_row, 0)

    row0 = pl.multiple_of(s * RPT, 8)
    for k in range(RPT // ZB):
        pltpu.sync_copy(zbuf, agg_sh.at[pl.ds(row0 + k * ZB, ZB)])
    plsc.subcore_barrier()

    def step(i, carry):
        base = pl.multiple_of(wid * EPW + i * C, 8)
        pltpu.sync_copy(src.at[pl.ds(base, C)], idx_s)
        pltpu.sync_copy(dst.at[pl.ds(base, C)], idx_d)
        gat = pltpu.async_copy(x.at[idx_s], rows, sem)
        for j in range(C // 16):
            dv = idx_d[pl.ds(j * 16, 16)]
            plsc.addupdate_scatter(hist, [dv], ones16)
        gat.wait()
        pltpu.sync_copy(rows, agg_sh.at[idx_d], add=True)
        return carry

    lax.fori_loop(0, NCHUNK, step, 0)
    plsc.subcore_barrier()

    @pl.when(s < 15)
    def _copy_full():
        pltpu.sync_copy(
            agg_sh.at[pl.ds(row0, RPT)],
            agg_out.at[c, pl.ds(row0, RPT)],
        )

    @pl.when(s == 15)
    def _copy_tail():
        pltpu.sync_copy(
            agg_sh.at[pl.ds(15 * RPT, N - 15 * RPT)],
            agg_out.at[c, pl.ds(15 * RPT, N - 15 * RPT)],
        )

    pltpu.sync_copy(
        hist.at[pl.ds(0, N)],
        deg_out.at[pl.ds(pl.multiple_of(wid * N, 8), N)],
    )


_sc_aggregate = pl.kernel(
    _sc_aggregate_body,
    out_type=(
        jax.ShapeDtypeStruct((2, N, D), jnp.float32),
        jax.ShapeDtypeStruct((NW * N,), jnp.float32),
    ),
    mesh=plsc.VectorSubcoreMesh(core_axis_name="c", subcore_axis_name="s"),
    compiler_params=pltpu.CompilerParams(needs_layout_passes=False),
    scratch_types=[
        pltpu.VMEM((C,), jnp.int32),
        pltpu.VMEM((C,), jnp.int32),
        pltpu.VMEM((C, D), jnp.float32),
        pltpu.VMEM((ZB, D), jnp.float32),
        pltpu.VMEM((NP,), jnp.float32),
        pltpu.VMEM_SHARED((NP, D), jnp.float32),
        pltpu.SemaphoreType.DMA,
    ],
)


def _selu(x):
    neg = _SELU_ALPHA * (jnp.exp(jnp.minimum(x, 0.0)) - 1.0)
    return _SELU_SCALE * jnp.where(x > 0, x, neg)


def _tc_layer_body(agg_ref, deg_ref, w_ref, b_ref, out_ref):
    a = agg_ref[0] + agg_ref[1]
    deg = jnp.maximum(jnp.sum(deg_ref[:], axis=1, keepdims=True), 1.0)
    y = jnp.dot(a, w_ref[:], preferred_element_type=jnp.float32, precision=lax.Precision.HIGHEST)
    out_ref[:] = _selu(y / deg + b_ref[:])


def _tc_final_body(agg_ref, deg_ref, w2_ref, b2_ref, w3_ref, b3_ref, out_ref):
    a = agg_ref[0] + agg_ref[1]
    deg = jnp.maximum(jnp.sum(deg_ref[:], axis=1, keepdims=True), 1.0)
    y = jnp.dot(a, w2_ref[:], preferred_element_type=jnp.float32, precision=lax.Precision.HIGHEST)
    h = _selu(y / deg + b2_ref[:])
    out_ref[:] = jnp.dot(h, w3_ref[:], preferred_element_type=jnp.float32, precision=lax.Precision.HIGHEST) + b3_ref[:]


_TC_BLOCK = 1000


def _tc_layer(agg, deg_t, w, b):
    return pl.pallas_call(
        _tc_layer_body,
        grid=(N // _TC_BLOCK,),
        in_specs=[
            pl.BlockSpec((2, _TC_BLOCK, D), lambda i: (0, i, 0)),
            pl.BlockSpec((_TC_BLOCK, NW), lambda i: (i, 0)),
            pl.BlockSpec((D, D), lambda i: (0, 0)),
            pl.BlockSpec((1, D), lambda i: (0, 0)),
        ],
        out_specs=pl.BlockSpec((_TC_BLOCK, D), lambda i: (i, 0)),
        out_shape=jax.ShapeDtypeStruct((N, D), jnp.float32),
    )(agg, deg_t, w, b.reshape(1, D))


def _tc_final(agg, deg_t, w2, b2, w3, b3):
    return pl.pallas_call(
        _tc_final_body,
        grid=(N // _TC_BLOCK,),
        in_specs=[
            pl.BlockSpec((2, _TC_BLOCK, D), lambda i: (0, i, 0)),
            pl.BlockSpec((_TC_BLOCK, NW), lambda i: (i, 0)),
            pl.BlockSpec((D, D), lambda i: (0, 0)),
            pl.BlockSpec((1, D), lambda i: (0, 0)),
            pl.BlockSpec((D, D), lambda i: (0, 0)),
            pl.BlockSpec((1, D), lambda i: (0, 0)),
        ],
        out_specs=pl.BlockSpec((_TC_BLOCK, D), lambda i: (i, 0)),
        out_shape=jax.ShapeDtypeStruct((N, D), jnp.float32),
    )(agg, deg_t, w2, b2.reshape(1, D), w3, b3.reshape(1, D))


@jax.jit
def kernel(x, edge_index, W1, b1, W2, b2, W3, b3):
    src = edge_index[0]
    dst = edge_index[1]
    agg1, deg1 = _sc_aggregate(x, src, dst)
    deg_t = deg1.reshape(NW, N).T
    h1 = _tc_layer(agg1, deg_t, W1, b1)
    agg2, _ = _sc_aggregate(h1, src, dst)
    return _tc_final(agg2, deg_t, W2, b2, W3, b3)


# speedup vs baseline: 11.2268x; 2.0585x over previous
import jax
import jax.numpy as jnp
from jax import lax
from jax.experimental import pallas as pl
from jax.experimental.pallas import tpu as pltpu
from jax.experimental.pallas import tpu_sc as plsc

N = 10000
E = 320000
D = 128
NW = 32
EPW = E // NW
C = 80
NCHUNK = EPW // C
NP = 10112
RPT = NP // 16
NBUF = 3
NGROUP = 5
G = NCHUNK // NGROUP

_SELU_ALPHA = 1.6732632423543772
_SELU_SCALE = 1.0507009873554805


def _sc_aggregate_body(x, src5, dst5, zeros_in, agg_out, deg_out,
                       sidx, didx, rows0, rows1, rows2, hist, agg_sh,
                       gs0, gs1, gs2, ss0, ss1, ss2):
    rows = (rows0, rows1, rows2)
    gsem = (gs0, gs1, gs2)
    ssem = (ss0, ss1, ss2)
    c = lax.axis_index("c")
    s = lax.axis_index("s")
    wid = c * 16 + s
    ones16 = jnp.ones((16,), jnp.float32)
    zeros16 = jnp.zeros((16,), jnp.float32)

    def zero_hist(i, carry):
        hist[pl.ds(pl.multiple_of(i * 16, 16), 16)] = zeros16
        return carry

    lax.fori_loop(0, NP // 16, zero_hist, 0)

    row0 = pl.multiple_of(s * RPT, 8)
    pltpu.sync_copy(zeros_in, agg_sh.at[pl.ds(row0, RPT)])
    plsc.subcore_barrier()

    for g in range(NGROUP):
        pltpu.sync_copy(src5.at[wid, g], sidx)
        pltpu.sync_copy(dst5.at[wid, g], didx)
        pending = {}
        for p in range(NBUF):
            pltpu.async_copy(x.at[sidx.at[p]], rows[p], gsem[p])
        for i in range(G):
            b = i % NBUF
            pltpu.make_async_copy(x.at[sidx.at[i]], rows[b], gsem[b]).wait()
            d = pltpu.make_async_copy(rows[b], agg_sh.at[didx.at[i]], ssem[b])
            d.start(add=True)
            pending[b] = d
            for k in range(C // 16):
                dv = didx[i, pl.ds(k * 16, 16)]
                plsc.addupdate_scatter(hist, [dv], ones16)
            if i >= 1:
                bb = (i - 1) % NBUF
                pending[bb].wait()
                if (i - 1) + NBUF < G:
                    pltpu.async_copy(
                        x.at[sidx.at[(i - 1) + NBUF]], rows[bb], gsem[bb])
        pending[(G - 1) % NBUF].wait()
    plsc.subcore_barrier()

    @pl.when(s < 15)
    def _copy_full():
        pltpu.sync_copy(
            agg_sh.at[pl.ds(row0, RPT)],
            agg_out.at[c, pl.ds(row0, RPT)],
        )

    @pl.when(s == 15)
    def _copy_tail():
        pltpu.sync_copy(
            agg_sh.at[pl.ds(15 * RPT, N - 15 * RPT)],
            agg_out.at[c, pl.ds(15 * RPT, N - 15 * RPT)],
        )

    pltpu.sync_copy(
        hist.at[pl.ds(0, N)],
        deg_out.at[pl.ds(pl.multiple_of(wid * N, 8), N)],
    )


_sc_aggregate = pl.kernel(
    _sc_aggregate_body,
    out_type=(
        jax.ShapeDtypeStruct((2, N, D), jnp.float32),
        jax.ShapeDtypeStruct((NW * N,), jnp.float32),
    ),
    mesh=plsc.VectorSubcoreMesh(core_axis_name="c", subcore_axis_name="s"),
    compiler_params=pltpu.CompilerParams(needs_layout_passes=False),
    scratch_types=(
        [
            pltpu.VMEM((G, C), jnp.int32),
            pltpu.VMEM((G, C), jnp.int32),
        ]
        + [pltpu.VMEM((C, D), jnp.float32)] * NBUF
        + [
            pltpu.VMEM((NP,), jnp.float32),
            pltpu.VMEM_SHARED((NP, D), jnp.float32),
        ]
        + [pltpu.SemaphoreType.DMA] * (2 * NBUF)
    ),
)


def _selu(x):
    neg = _SELU_ALPHA * (jnp.exp(jnp.minimum(x, 0.0)) - 1.0)
    return _SELU_SCALE * jnp.where(x > 0, x, neg)


def _tc_layer_body(agg_ref, deg_ref, w_ref, b_ref, out_ref):
    a = agg_ref[0] + agg_ref[1]
    deg = jnp.maximum(jnp.sum(deg_ref[:], axis=1, keepdims=True), 1.0)
    y = jnp.dot(a, w_ref[:], preferred_element_type=jnp.float32, precision=lax.Precision.HIGHEST)
    out_ref[:] = _selu(y / deg + b_ref[:])


def _tc_final_body(agg_ref, deg_ref, w2_ref, b2_ref, w3_ref, b3_ref, out_ref):
    a = agg_ref[0] + agg_ref[1]
    deg = jnp.maximum(jnp.sum(deg_ref[:], axis=1, keepdims=True), 1.0)
    y = jnp.dot(a, w2_ref[:], preferred_element_type=jnp.float32, precision=lax.Precision.HIGHEST)
    h = _selu(y / deg + b2_ref[:])
    out_ref[:] = jnp.dot(h, w3_ref[:], preferred_element_type=jnp.float32, precision=lax.Precision.HIGHEST) + b3_ref[:]


_TC_BLOCK = 1000


def _tc_layer(agg, deg_t, w, b):
    return pl.pallas_call(
        _tc_layer_body,
        grid=(N // _TC_BLOCK,),
        in_specs=[
            pl.BlockSpec((2, _TC_BLOCK, D), lambda i: (0, i, 0)),
            pl.BlockSpec((_TC_BLOCK, NW), lambda i: (i, 0)),
            pl.BlockSpec((D, D), lambda i: (0, 0)),
            pl.BlockSpec((1, D), lambda i: (0, 0)),
        ],
        out_specs=pl.BlockSpec((_TC_BLOCK, D), lambda i: (i, 0)),
        out_shape=jax.ShapeDtypeStruct((N, D), jnp.float32),
    )(agg, deg_t, w, b.reshape(1, D))


def _tc_final(agg, deg_t, w2, b2, w3, b3):
    return pl.pallas_call(
        _tc_final_body,
        grid=(N // _TC_BLOCK,),
        in_specs=[
            pl.BlockSpec((2, _TC_BLOCK, D), lambda i: (0, i, 0)),
            pl.BlockSpec((_TC_BLOCK, NW), lambda i: (i, 0)),
            pl.BlockSpec((D, D), lambda i: (0, 0)),
            pl.BlockSpec((1, D), lambda i: (0, 0)),
            pl.BlockSpec((D, D), lambda i: (0, 0)),
            pl.BlockSpec((1, D), lambda i: (0, 0)),
        ],
        out_specs=pl.BlockSpec((_TC_BLOCK, D), lambda i: (i, 0)),
        out_shape=jax.ShapeDtypeStruct((N, D), jnp.float32),
    )(agg, deg_t, w2, b2.reshape(1, D), w3, b3.reshape(1, D))


@jax.jit
def kernel(x, edge_index, W1, b1, W2, b2, W3, b3):
    ei5 = edge_index.reshape(2, NW, NGROUP, G, C)
    src5 = ei5[0]
    dst5 = ei5[1]
    zeros_in = jnp.zeros((RPT, D), jnp.float32)
    agg1, deg1 = _sc_aggregate(x, src5, dst5, zeros_in)
    deg_t = deg1.reshape(NW, N).T
    h1 = _tc_layer(agg1, deg_t, W1, b1)
    agg2, _ = _sc_aggregate(h1, src5, dst5, zeros_in)
    return _tc_final(agg2, deg_t, W2, b2, W3, b3)
